# baseline (speedup 1.0000x reference)
import dataclasses
import functools

import numpy as np
import jax
import jax.numpy as jnp
from jax import lax
from jax.experimental import pallas as pl
from jax.experimental.pallas import tpu as pltpu
from jax.experimental.pallas import tpu_sc as plsc

N_NODES = 10000
N_EDGES = 320000
D = 128
D_EDGE = 16
BN_EPS = 1e-5

NC = 2
NS = 16
LANES = 16
NW = NC * NS
E_PER_W = N_EDGES // NW
E_BLK = 40
N_CHUNK = E_PER_W // E_BLK
AGG_ROWS = 10240
SROWS = AGG_ROWS // NS

def _sc_layer_agg(h, ep_i32, src, dst2):
    mesh = plsc.VectorSubcoreMesh(core_axis_name="c", subcore_axis_name="s")
    cp = pltpu.CompilerParams()
    if "needs_layout_passes" in pltpu.CompilerParams.__dataclass_fields__:
        cp = dataclasses.replace(cp, needs_layout_passes=False)

    @functools.partial(
        pl.kernel,
        out_type=jax.ShapeDtypeStruct((NC, N_NODES, D), jnp.float32),
        mesh=mesh,
        compiler_params=cp,
        scratch_types=[
            pltpu.VMEM((E_PER_W,), jnp.int32),
            pltpu.VMEM((1, E_BLK), jnp.int32),
            pltpu.VMEM((1, E_BLK), jnp.int32),
            pltpu.VMEM((E_BLK, D), jnp.float32),
            pltpu.VMEM((E_BLK, D), jnp.float32),
            pltpu.VMEM((E_BLK, D), jnp.int32),
            pltpu.VMEM((E_BLK, D), jnp.int32),
            pltpu.VMEM((E_BLK, D), jnp.float32),
            pltpu.VMEM((E_BLK, D), jnp.float32),
            pltpu.VMEM_SHARED((AGG_ROWS, D), jnp.float32),
            pltpu.SemaphoreType.DMA,
            pltpu.SemaphoreType.DMA,
            pltpu.SemaphoreType.DMA,
            pltpu.SemaphoreType.DMA,
            pltpu.SemaphoreType.DMA,
            pltpu.SemaphoreType.DMA,
            pltpu.SemaphoreType.DMA,
            pltpu.SemaphoreType.DMA,
        ],
    )
    def k(h_hbm, ep_hbm, src_hbm, dst_hbm, out_hbm,
          srci, dsti0, dsti1, rows0, rows1, epbA, epbB, msg0, msg1, agg,
          sg0, sg1, seA, seB, ss0, ss1, sd0, sd1):
        cid = lax.axis_index("c")
        sid = lax.axis_index("s")
        wid = cid * NS + sid
        ebase = wid * E_PER_W

        pltpu.sync_copy(src_hbm.at[pl.ds(ebase, E_PER_W)], srci)

        zv = jnp.zeros((LANES,), jnp.float32)

        @pl.loop(0, E_BLK)
        def _(r):
            for j in range(D // LANES):
                msg0[r, pl.ds(j * LANES, LANES)] = zv

        @pl.loop(0, SROWS // E_BLK)
        def _(kz):
            pltpu.sync_copy(
                msg0, agg.at[pl.ds(sid * SROWS + kz * E_BLK, E_BLK)])

        def g_desc(ci, rows, sg):
            off = pl.multiple_of(ci * E_BLK, 8)
            return pltpu.make_async_copy(
                h_hbm.at[srci.at[pl.ds(off, E_BLK)]], rows, sg)

        def ep_desc(ci, epb, se):
            epoff = pl.multiple_of((ebase + ci * E_BLK) // 2, 8)
            return pltpu.make_async_copy(
                ep_hbm.at[pl.ds(epoff, E_BLK)], epb, se)

        def d_desc(ci, dsti, sd):
            return pltpu.make_async_copy(
                dst_hbm.at[wid, pl.ds(ci, 1)], dsti, sd)

        def compute(kk, rows, epb, msg):
            @pl.loop(0, E_BLK // 2)
            def _(q):
                for gg in range(D // LANES):
                    w16 = epb[kk * (E_BLK // 2) + q, pl.ds(LANES * gg, LANES)]
                    ea, eb = plsc.unpack(
                        plsc.bitcast(w16, jnp.bfloat16),
                        format=plsc.PackFormat.INTERLEAVED,
                        preferred_element_type=jnp.float32)
                    s = pl.ds(LANES * gg, LANES)
                    msg[2 * q, s] = jnp.maximum(rows[2 * q, s] + ea, 0.0)
                    msg[2 * q + 1, s] = jnp.maximum(rows[2 * q + 1, s] + eb, 0.0)

        def do_chunk(ci, kk, rows, epb, msg, dsti, sg, sd, ss):
            g_desc(ci, rows, sg).wait()
            d_desc(ci, dsti, sd).wait()
            compute(kk, rows, epb, msg)
            sc = pltpu.make_async_copy(msg, agg.at[dsti.at[0]], ss)
            sc.start(add=True)
            return sc

        d_desc(0, dsti0, sd0).start()
        d_desc(1, dsti1, sd1).start()
        g_desc(0, rows0, sg0).start()
        g_desc(1, rows1, sg1).start()
        ep_desc(0, epbA, seA).start()
        ep_desc(2, epbB, seB).start()

        plsc.subcore_barrier()

        @pl.loop(0, N_CHUNK // 4)
        def _(j):
            c0 = j * 4

            ep_desc(c0, epbA, seA).wait()
            scA0 = do_chunk(c0, 0, rows0, epbA, msg0, dsti0, sg0, sd0, ss0)
            g_desc(c0 + 2, rows0, sg0).start()
            scA1 = do_chunk(c0 + 1, 1, rows1, epbA, msg1, dsti1, sg1, sd1, ss1)
            g_desc(c0 + 3, rows1, sg1).start()
            ep_desc(c0 + 4, epbA, seA).start()
            scA0.wait()
            d_desc(c0 + 2, dsti0, sd0).start()
            scA1.wait()
            d_desc(c0 + 3, dsti1, sd1).start()

            ep_desc(c0 + 2, epbB, seB).wait()
            scB0 = do_chunk(c0 + 2, 0, rows0, epbB, msg0, dsti0, sg0, sd0, ss0)
            g_desc(c0 + 4, rows0, sg0).start()
            scB1 = do_chunk(c0 + 3, 1, rows1, epbB, msg1, dsti1, sg1, sd1, ss1)
            g_desc(c0 + 5, rows1, sg1).start()

            @pl.when(c0 + 6 < N_CHUNK)
            def _():
                ep_desc(c0 + 6, epbB, seB).start()

            scB0.wait()
            d_desc(c0 + 4, dsti0, sd0).start()
            scB1.wait()
            d_desc(c0 + 5, dsti1, sd1).start()

        ct = N_CHUNK - 2
        ep_desc(ct, epbA, seA).wait()
        scT0 = do_chunk(ct, 0, rows0, epbA, msg0, dsti0, sg0, sd0, ss0)
        scT1 = do_chunk(ct + 1, 1, rows1, epbA, msg1, dsti1, sg1, sd1, ss1)
        scT0.wait()
        scT1.wait()

        plsc.subcore_barrier()

        @pl.when(sid < NS - 1)
        def _():
            pltpu.sync_copy(
                agg.at[pl.ds(sid * SROWS, SROWS)],
                out_hbm.at[cid, pl.ds(sid * SROWS, SROWS)])

        @pl.when(sid == NS - 1)
        def _():
            pltpu.sync_copy(
                agg.at[pl.ds((NS - 1) * SROWS, N_NODES - (NS - 1) * SROWS)],
                out_hbm.at[cid, pl.ds((NS - 1) * SROWS, N_NODES - (NS - 1) * SROWS)])

    return k(h, ep_i32, src, dst2)


_EP_ROWS = 4000


def _edge_proj(edge_attr, We):
    def body(ea_ref, w_ref, o_ref):
        s = jnp.dot(
            ea_ref[...], w_ref[...], preferred_element_type=jnp.float32)
        o_ref[...] = pltpu.bitcast(s.astype(jnp.bfloat16), jnp.int32)

    return pl.pallas_call(
        body,
        grid=(N_EDGES // _EP_ROWS,),
        in_specs=[pl.BlockSpec((_EP_ROWS, D_EDGE), lambda i: (i, 0)),
                  pl.BlockSpec((D_EDGE, D), lambda i: (0, 0))],
        out_specs=pl.BlockSpec((_EP_ROWS // 2, D), lambda i: (i, 0)),
        out_shape=jax.ShapeDtypeStruct((N_EDGES // 2, D), jnp.int32),
    )(edge_attr, We)


def _bn_relu_res(h, agg_ref, w_ref):
    s = jnp.dot(h + agg_ref[0] + agg_ref[1], w_ref[...],
                preferred_element_type=jnp.float32)
    mu = jnp.mean(s, axis=0, keepdims=True)
    var = jnp.mean((s - mu) ** 2, axis=0, keepdims=True)
    hn = (s - mu) * lax.rsqrt(var + BN_EPS)
    return jnp.maximum(hn, 0.0) + h


def _node_update(h, agg, W):
    def body(h_ref, a_ref, w_ref, o_ref):
        o_ref[...] = _bn_relu_res(h_ref[...], a_ref, w_ref)

    return pl.pallas_call(
        body,
        out_shape=jax.ShapeDtypeStruct((N_NODES, D), jnp.float32),
    )(h, agg, W)


def _node_update_final(h, agg, W, Wout, bout2):
    def body(h_ref, a_ref, w_ref, wo_ref, b_ref, o_ref):
        hn = _bn_relu_res(h_ref[...], a_ref, w_ref)
        o_ref[...] = jnp.maximum(
            jnp.dot(hn, wo_ref[...], preferred_element_type=jnp.float32)
            + b_ref[...], 0.0)

    return pl.pallas_call(
        body,
        out_shape=jax.ShapeDtypeStruct((N_NODES, D), jnp.float32),
    )(h, agg, W, Wout, bout2)


def kernel(x, edge_index, edge_attr, batch, We0, W0, We1, W1, We2, W2, Wout, bout):
    src = edge_index[0].astype(jnp.int32)
    dst2 = edge_index[1].astype(jnp.int32).reshape(NW, N_CHUNK, E_BLK)
    ep0 = _edge_proj(edge_attr, We0)
    ep1 = _edge_proj(edge_attr, We1)
    ep2 = _edge_proj(edge_attr, We2)

    h = x
    agg = _sc_layer_agg(h, ep0, src, dst2)
    h = _node_update(h, agg, W0)
    agg = _sc_layer_agg(h, ep1, src, dst2)
    h = _node_update(h, agg, W1)
    agg = _sc_layer_agg(h, ep2, src, dst2)
    return _node_update_final(h, agg, W2, Wout, jnp.reshape(bout, (1, D)))


# speedup vs baseline: 1.2865x; 1.2865x over previous
import dataclasses
import functools

import jax
import jax.numpy as jnp
from jax import lax
from jax.experimental import pallas as pl
from jax.experimental.pallas import tpu as pltpu
from jax.experimental.pallas import tpu_sc as plsc

N_NODES = 10000
N_EDGES = 320000
D = 128
D_EDGE = 16
BN_EPS = 1e-5

NC = 2
NS = 16
LANES = 16
NW = NC * NS
E_PER_W = N_EDGES // NW
E_BLK = 40
N_CHUNK = E_PER_W // E_BLK
AGG_ROWS = 10240
SROWS = AGG_ROWS // NS

def _sc_layer_agg(h, ep, src, dst2):
    mesh = plsc.VectorSubcoreMesh(core_axis_name="c", subcore_axis_name="s")
    cp = pltpu.CompilerParams()
    if "needs_layout_passes" in pltpu.CompilerParams.__dataclass_fields__:
        cp = dataclasses.replace(cp, needs_layout_passes=False)

    @functools.partial(
        pl.kernel,
        out_type=jax.ShapeDtypeStruct((NC, N_NODES, D), jnp.float32),
        mesh=mesh,
        compiler_params=cp,
        scratch_types=[
            pltpu.VMEM((E_PER_W,), jnp.int32),
            pltpu.VMEM((1, E_BLK), jnp.int32),
            pltpu.VMEM((1, E_BLK), jnp.int32),
            pltpu.VMEM((E_BLK, D), jnp.float32),
            pltpu.VMEM((E_BLK, D), jnp.float32),
            pltpu.VMEM((E_BLK, D), jnp.float32),
            pltpu.VMEM((E_BLK, D), jnp.float32),
            pltpu.VMEM((E_BLK, D), jnp.float32),
            pltpu.VMEM((E_BLK, D), jnp.float32),
            pltpu.VMEM_SHARED((AGG_ROWS, D), jnp.float32),
            pltpu.SemaphoreType.DMA,
            pltpu.SemaphoreType.DMA,
            pltpu.SemaphoreType.DMA,
            pltpu.SemaphoreType.DMA,
            pltpu.SemaphoreType.DMA,
            pltpu.SemaphoreType.DMA,
            pltpu.SemaphoreType.DMA,
            pltpu.SemaphoreType.DMA,
        ],
    )
    def k(h_hbm, ep_hbm, src_hbm, dst_hbm, out_hbm,
          srci, dsti0, dsti1, rows0, rows1, epb0, epb1, msg0, msg1, agg,
          sg0, sg1, se0, se1, ss0, ss1, sd0, sd1):
        cid = lax.axis_index("c")
        sid = lax.axis_index("s")
        wid = cid * NS + sid
        ebase = wid * E_PER_W

        pltpu.sync_copy(src_hbm.at[pl.ds(ebase, E_PER_W)], srci)

        zv = jnp.zeros((LANES,), jnp.float32)

        @pl.loop(0, E_BLK)
        def _(r):
            for j in range(D // LANES):
                msg0[r, pl.ds(j * LANES, LANES)] = zv

        @pl.loop(0, SROWS // E_BLK)
        def _(kz):
            pltpu.sync_copy(
                msg0, agg.at[pl.ds(sid * SROWS + kz * E_BLK, E_BLK)])

        def issue_ge(ci, rows, epb, sg, se):
            off = pl.multiple_of(ci * E_BLK, 8)
            pltpu.make_async_copy(
                h_hbm.at[srci.at[pl.ds(off, E_BLK)]], rows, sg).start()
            pltpu.make_async_copy(
                ep_hbm.at[pl.ds(ebase + off, E_BLK)], epb, se).start()

        def issue_d(ci, dsti, sd):
            pltpu.make_async_copy(
                dst_hbm.at[wid, pl.ds(ci, 1)], dsti, sd).start()

        def issue_in(ci, rows, epb, dsti, sg, se, sd):
            issue_ge(ci, rows, epb, sg, se)
            issue_d(ci, dsti, sd)

        def wait_in(ci, rows, epb, dsti, sg, se, sd):
            off = pl.multiple_of(ci * E_BLK, 8)
            pltpu.make_async_copy(
                h_hbm.at[srci.at[pl.ds(off, E_BLK)]], rows, sg).wait()
            pltpu.make_async_copy(
                ep_hbm.at[pl.ds(ebase + off, E_BLK)], epb, se).wait()
            pltpu.make_async_copy(
                dst_hbm.at[wid, pl.ds(ci, 1)], dsti, sd).wait()

        def compute(rows, epb, msg):
            @pl.loop(0, E_BLK)
            def _(e):
                for j in range(D // LANES):
                    sl = pl.ds(j * LANES, LANES)
                    msg[e, sl] = jnp.maximum(epb[e, sl] + rows[e, sl], 0.0)

        issue_in(0, rows0, epb0, dsti0, sg0, se0, sd0)
        issue_in(1, rows1, epb1, dsti1, sg1, se1, sd1)

        plsc.subcore_barrier()

        @pl.loop(0, N_CHUNK // 2)
        def _(i):
            a = i * 2
            b = a + 1
            wait_in(a, rows0, epb0, dsti0, sg0, se0, sd0)
            compute(rows0, epb0, msg0)
            sc_a = pltpu.make_async_copy(msg0, agg.at[dsti0.at[0]], ss0)
            sc_a.start(add=True)

            @pl.when(a + 2 < N_CHUNK)
            def _():
                issue_ge(a + 2, rows0, epb0, sg0, se0)

            wait_in(b, rows1, epb1, dsti1, sg1, se1, sd1)
            compute(rows1, epb1, msg1)
            sc_b = pltpu.make_async_copy(msg1, agg.at[dsti1.at[0]], ss1)
            sc_b.start(add=True)

            @pl.when(b + 2 < N_CHUNK)
            def _():
                issue_ge(b + 2, rows1, epb1, sg1, se1)

            sc_a.wait()

            @pl.when(a + 2 < N_CHUNK)
            def _():
                issue_d(a + 2, dsti0, sd0)

            sc_b.wait()

            @pl.when(b + 2 < N_CHUNK)
            def _():
                issue_d(b + 2, dsti1, sd1)

        plsc.subcore_barrier()

        @pl.when(sid < NS - 1)
        def _():
            pltpu.sync_copy(
                agg.at[pl.ds(sid * SROWS, SROWS)],
                out_hbm.at[cid, pl.ds(sid * SROWS, SROWS)])

        @pl.when(sid == NS - 1)
        def _():
            pltpu.sync_copy(
                agg.at[pl.ds((NS - 1) * SROWS, N_NODES - (NS - 1) * SROWS)],
                out_hbm.at[cid, pl.ds((NS - 1) * SROWS, N_NODES - (NS - 1) * SROWS)])

    return k(h, ep, src, dst2)


_EP_ROWS = 4000


def _edge_proj(edge_attr, We):
    def body(ea_ref, w_ref, o_ref):
        o_ref[...] = jnp.dot(
            ea_ref[...], w_ref[...], preferred_element_type=jnp.float32)

    return pl.pallas_call(
        body,
        grid=(N_EDGES // _EP_ROWS,),
        in_specs=[pl.BlockSpec((_EP_ROWS, D_EDGE), lambda i: (i, 0)),
                  pl.BlockSpec((D_EDGE, D), lambda i: (0, 0))],
        out_specs=pl.BlockSpec((_EP_ROWS, D), lambda i: (i, 0)),
        out_shape=jax.ShapeDtypeStruct((N_EDGES, D), jnp.float32),
    )(edge_attr, We)


def _bn_relu_res(h, agg_ref, w_ref):
    s = jnp.dot(h + agg_ref[0] + agg_ref[1], w_ref[...],
                preferred_element_type=jnp.float32)
    mu = jnp.mean(s, axis=0, keepdims=True)
    var = jnp.mean((s - mu) ** 2, axis=0, keepdims=True)
    hn = (s - mu) * lax.rsqrt(var + BN_EPS)
    return jnp.maximum(hn, 0.0) + h


def _node_update(h, agg, W):
    def body(h_ref, a_ref, w_ref, o_ref):
        o_ref[...] = _bn_relu_res(h_ref[...], a_ref, w_ref)

    return pl.pallas_call(
        body,
        out_shape=jax.ShapeDtypeStruct((N_NODES, D), jnp.float32),
    )(h, agg, W)


def _node_update_final(h, agg, W, Wout, bout2):
    def body(h_ref, a_ref, w_ref, wo_ref, b_ref, o_ref):
        hn = _bn_relu_res(h_ref[...], a_ref, w_ref)
        o_ref[...] = jnp.maximum(
            jnp.dot(hn, wo_ref[...], preferred_element_type=jnp.float32)
            + b_ref[...], 0.0)

    return pl.pallas_call(
        body,
        out_shape=jax.ShapeDtypeStruct((N_NODES, D), jnp.float32),
    )(h, agg, W, Wout, bout2)


def kernel(x, edge_index, edge_attr, batch, We0, W0, We1, W1, We2, W2, Wout, bout):
    src = edge_index[0].astype(jnp.int32)
    dst2 = edge_index[1].astype(jnp.int32).reshape(NW, N_CHUNK, E_BLK)
    ep0 = _edge_proj(edge_attr, We0)
    ep1 = _edge_proj(edge_attr, We1)
    ep2 = _edge_proj(edge_attr, We2)

    h = x
    agg = _sc_layer_agg(h, ep0, src, dst2)
    h = _node_update(h, agg, W0)
    agg = _sc_layer_agg(h, ep1, src, dst2)
    h = _node_update(h, agg, W1)
    agg = _sc_layer_agg(h, ep2, src, dst2)
    return _node_update_final(h, agg, W2, Wout, jnp.reshape(bout, (1, D)))


# speedup vs baseline: 1.3313x; 1.0348x over previous
import dataclasses
import functools

import jax
import jax.numpy as jnp
from jax import lax
from jax.experimental import pallas as pl
from jax.experimental.pallas import tpu as pltpu
from jax.experimental.pallas import tpu_sc as plsc

N_NODES = 10000
N_EDGES = 320000
D = 128
D_EDGE = 16
BN_EPS = 1e-5

NC = 2
NS = 16
LANES = 16
NW = NC * NS
E_PER_W = N_EDGES // NW
E_BLK = 40
N_CHUNK = E_PER_W // E_BLK
AGG_ROWS = 10240
SROWS = AGG_ROWS // NS

def _sc_layer_agg(h, ep, src, dst2):
    mesh = plsc.VectorSubcoreMesh(core_axis_name="c", subcore_axis_name="s")
    cp = pltpu.CompilerParams()
    if "needs_layout_passes" in pltpu.CompilerParams.__dataclass_fields__:
        cp = dataclasses.replace(cp, needs_layout_passes=False)

    @functools.partial(
        pl.kernel,
        out_type=jax.ShapeDtypeStruct((NC, N_NODES, D), jnp.float32),
        mesh=mesh,
        compiler_params=cp,
        scratch_types=[
            pltpu.VMEM((E_PER_W,), jnp.int32),
            pltpu.VMEM((1, E_BLK), jnp.int32),
            pltpu.VMEM((1, E_BLK), jnp.int32),
            pltpu.VMEM((E_BLK, D), jnp.float32),
            pltpu.VMEM((E_BLK, D), jnp.float32),
            pltpu.VMEM((E_BLK, D), jnp.float32),
            pltpu.VMEM((E_BLK, D), jnp.float32),
            pltpu.VMEM((E_BLK, D), jnp.float32),
            pltpu.VMEM((E_BLK, D), jnp.float32),
            pltpu.VMEM_SHARED((AGG_ROWS, D), jnp.float32),
            pltpu.SemaphoreType.DMA,
            pltpu.SemaphoreType.DMA,
            pltpu.SemaphoreType.DMA,
            pltpu.SemaphoreType.DMA,
            pltpu.SemaphoreType.DMA,
            pltpu.SemaphoreType.DMA,
            pltpu.SemaphoreType.DMA,
            pltpu.SemaphoreType.DMA,
        ],
    )
    def k(h_hbm, ep_hbm, src_hbm, dst_hbm, out_hbm,
          srci, dsti0, dsti1, rows0, rows1, epb0, epb1, msg0, msg1, agg,
          sg0, sg1, se0, se1, ss0, ss1, sd0, sd1):
        cid = lax.axis_index("c")
        sid = lax.axis_index("s")
        wid = cid * NS + sid
        ebase = wid * E_PER_W

        pltpu.sync_copy(src_hbm.at[pl.ds(ebase, E_PER_W)], srci)

        zv = jnp.zeros((LANES,), jnp.float32)

        @pl.loop(0, E_BLK)
        def _(r):
            for j in range(D // LANES):
                msg0[r, pl.ds(j * LANES, LANES)] = zv

        @pl.loop(0, SROWS // E_BLK)
        def _(kz):
            pltpu.sync_copy(
                msg0, agg.at[pl.ds(sid * SROWS + kz * E_BLK, E_BLK)])

        def issue_ge(ci, rows, epb, sg, se):
            off = pl.multiple_of(ci * E_BLK, 8)
            pltpu.make_async_copy(
                h_hbm.at[srci.at[pl.ds(off, E_BLK)]], rows, sg).start()
            pltpu.make_async_copy(
                ep_hbm.at[pl.ds(ebase + off, E_BLK)], epb, se).start()

        def issue_d(ci, dsti, sd):
            pltpu.make_async_copy(
                dst_hbm.at[wid, pl.ds(ci, 1)], dsti, sd).start()

        def issue_in(ci, rows, epb, dsti, sg, se, sd):
            issue_ge(ci, rows, epb, sg, se)
            issue_d(ci, dsti, sd)

        def wait_in(ci, rows, epb, dsti, sg, se, sd):
            off = pl.multiple_of(ci * E_BLK, 8)
            pltpu.make_async_copy(
                h_hbm.at[srci.at[pl.ds(off, E_BLK)]], rows, sg).wait()
            pltpu.make_async_copy(
                ep_hbm.at[pl.ds(ebase + off, E_BLK)], epb, se).wait()
            pltpu.make_async_copy(
                dst_hbm.at[wid, pl.ds(ci, 1)], dsti, sd).wait()

        def compute(rows, epb, msg):
            @pl.loop(0, E_BLK)
            def _(e):
                for j in range(D // LANES):
                    sl = pl.ds(j * LANES, LANES)
                    msg[e, sl] = jnp.maximum(epb[e, sl] + rows[e, sl], 0.0)

        issue_in(0, rows0, epb0, dsti0, sg0, se0, sd0)
        issue_in(1, rows1, epb1, dsti1, sg1, se1, sd1)

        plsc.subcore_barrier()

        @pl.loop(0, N_CHUNK // 2)
        def _(i):
            a = i * 2
            b = a + 1
            wait_in(a, rows0, epb0, dsti0, sg0, se0, sd0)
            compute(rows0, epb0, msg0)
            sc_a = pltpu.make_async_copy(msg0, agg.at[dsti0.at[0]], ss0)
            sc_a.start(add=True)

            @pl.when(a + 2 < N_CHUNK)
            def _():
                issue_ge(a + 2, rows0, epb0, sg0, se0)

            wait_in(b, rows1, epb1, dsti1, sg1, se1, sd1)
            compute(rows1, epb1, msg1)
            sc_b = pltpu.make_async_copy(msg1, agg.at[dsti1.at[0]], ss1)
            sc_b.start(add=True)

            @pl.when(b + 2 < N_CHUNK)
            def _():
                issue_ge(b + 2, rows1, epb1, sg1, se1)

            sc_a.wait()

            @pl.when(a + 2 < N_CHUNK)
            def _():
                issue_d(a + 2, dsti0, sd0)

            sc_b.wait()

            @pl.when(b + 2 < N_CHUNK)
            def _():
                issue_d(b + 2, dsti1, sd1)

        plsc.subcore_barrier()

        @pl.when(sid < NS - 1)
        def _():
            pltpu.sync_copy(
                agg.at[pl.ds(sid * SROWS, SROWS)],
                out_hbm.at[cid, pl.ds(sid * SROWS, SROWS)])

        @pl.when(sid == NS - 1)
        def _():
            pltpu.sync_copy(
                agg.at[pl.ds((NS - 1) * SROWS, N_NODES - (NS - 1) * SROWS)],
                out_hbm.at[cid, pl.ds((NS - 1) * SROWS, N_NODES - (NS - 1) * SROWS)])

    return k(h, ep, src, dst2)


_EP_ROWS = 4000


def _edge_proj(edge_attr, We0, We1, We2):
    def body(ea_ref, w0_ref, w1_ref, w2_ref, o0_ref, o1_ref, o2_ref):
        ea = ea_ref[...]
        o0_ref[...] = jnp.dot(ea, w0_ref[...], preferred_element_type=jnp.float32)
        o1_ref[...] = jnp.dot(ea, w1_ref[...], preferred_element_type=jnp.float32)
        o2_ref[...] = jnp.dot(ea, w2_ref[...], preferred_element_type=jnp.float32)

    w_spec = pl.BlockSpec((D_EDGE, D), lambda i: (0, 0))
    o_spec = pl.BlockSpec((_EP_ROWS, D), lambda i: (i, 0))
    return pl.pallas_call(
        body,
        grid=(N_EDGES // _EP_ROWS,),
        in_specs=[pl.BlockSpec((_EP_ROWS, D_EDGE), lambda i: (i, 0)),
                  w_spec, w_spec, w_spec],
        out_specs=[o_spec, o_spec, o_spec],
        out_shape=[jax.ShapeDtypeStruct((N_EDGES, D), jnp.float32)] * 3,
    )(edge_attr, We0, We1, We2)


def _bn_relu_res(h, agg_ref, w_ref):
    s = jnp.dot(h + agg_ref[0] + agg_ref[1], w_ref[...],
                preferred_element_type=jnp.float32)
    mu = jnp.mean(s, axis=0, keepdims=True)
    var = jnp.mean((s - mu) ** 2, axis=0, keepdims=True)
    hn = (s - mu) * lax.rsqrt(var + BN_EPS)
    return jnp.maximum(hn, 0.0) + h


def _node_update(h, agg, W):
    def body(h_ref, a_ref, w_ref, o_ref):
        o_ref[...] = _bn_relu_res(h_ref[...], a_ref, w_ref)

    return pl.pallas_call(
        body,
        out_shape=jax.ShapeDtypeStruct((N_NODES, D), jnp.float32),
    )(h, agg, W)


def _node_update_final(h, agg, W, Wout, bout2):
    def body(h_ref, a_ref, w_ref, wo_ref, b_ref, o_ref):
        hn = _bn_relu_res(h_ref[...], a_ref, w_ref)
        o_ref[...] = jnp.maximum(
            jnp.dot(hn, wo_ref[...], preferred_element_type=jnp.float32)
            + b_ref[...], 0.0)

    return pl.pallas_call(
        body,
        out_shape=jax.ShapeDtypeStruct((N_NODES, D), jnp.float32),
    )(h, agg, W, Wout, bout2)


def kernel(x, edge_index, edge_attr, batch, We0, W0, We1, W1, We2, W2, Wout, bout):
    src = edge_index[0].astype(jnp.int32)
    dst2 = edge_index[1].astype(jnp.int32).reshape(NW, N_CHUNK, E_BLK)
    ep0, ep1, ep2 = _edge_proj(edge_attr, We0, We1, We2)

    h = x
    agg = _sc_layer_agg(h, ep0, src, dst2)
    h = _node_update(h, agg, W0)
    agg = _sc_layer_agg(h, ep1, src, dst2)
    h = _node_update(h, agg, W1)
    agg = _sc_layer_agg(h, ep2, src, dst2)
    return _node_update_final(h, agg, W2, Wout, jnp.reshape(bout, (1, D)))
